# baseline (speedup 1.0000x reference)
import functools

import jax
import jax.numpy as jnp
import numpy as _np
from jax import lax
from jax.experimental import pallas as pl
from jax.experimental.pallas import tpu as pltpu, tpu_sc as plsc

N_ITERS = 3
LAMBDA_INIT = 0.01

_info = plsc.get_sparse_core_info()
_NC, _NS = _info.num_cores, _info.num_subcores
_NW = _NC * _NS
_L = 16


def _rhe_int(u):
    uh = u + 0.5
    i = uh.astype(jnp.int32)
    fi = i.astype(jnp.float32)
    fl = jnp.where(fi > uh, fi - 1.0, fi)
    ifl = fl.astype(jnp.int32)
    half = (fl - u) == 0.5
    odd = (ifl & 1) == 1
    return jnp.where(half & odd, ifl - 1, ifl)


def _build_gn_kernel(N, C, H, W, N_pad):
    HW = H * W
    PPT = N_pad // _NW
    G = PPT // _L
    CK = C // _L
    mesh = plsc.VectorSubcoreMesh(core_axis_name="c", subcore_axis_name="s")
    f32, i32 = jnp.float32, jnp.int32

    @functools.partial(
        pl.kernel, mesh=mesh,
        compiler_params=pltpu.CompilerParams(
            use_tc_tiling_on_sc=False, needs_layout_passes=False),
        out_type=jax.ShapeDtypeStruct((_NW, 28, _L), f32),
        scratch_types=[
            pltpu.VMEM((32,), f32),
            pltpu.VMEM((PPT,), f32),
            pltpu.VMEM((PPT,), f32),
            pltpu.VMEM((PPT,), f32),
            pltpu.VMEM((2, 80), i32),
            pltpu.VMEM((PPT,), f32),
            pltpu.VMEM((PPT,), f32),
            pltpu.VMEM((PPT,), f32),
            pltpu.VMEM((PPT, C), f32),
            pltpu.VMEM((PPT, C), f32),
            pltpu.VMEM((PPT, C), f32),
            pltpu.VMEM((PPT, C), f32),
            pltpu.VMEM((28, _L), f32),
            pltpu.SemaphoreType.DMA,
        ],
    )
    def gn_step(par_hbm, xs_hbm, ys_hbm, zs_hbm, fref_hbm, fmt_hbm, gxt_hbm,
                gyt_hbm, out_hbm, par_v, xs_v, ys_v, zs_v, idx_v, px_v, py_v,
                pz_v, fm_v, gx_v, gy_v, fr_v, st_v, sem):
        wid = lax.axis_index("s") * _NC + lax.axis_index("c")
        base = wid * PPT
        pltpu.sync_copy(par_hbm, par_v)
        pltpu.sync_copy(xs_hbm.at[pl.ds(base, PPT)], xs_v)
        pltpu.sync_copy(ys_hbm.at[pl.ds(base, PPT)], ys_v)
        pltpu.sync_copy(zs_hbm.at[pl.ds(base, PPT)], zs_v)
        pltpu.sync_copy(fref_hbm.at[pl.ds(base, PPT)], fr_v)

        p0 = par_v[pl.ds(0, _L)]
        p1 = par_v[pl.ds(_L, _L)]
        par = [p0[i] for i in range(_L)] + [p1[i] for i in range(5)]
        (R00, R01, R02, R10, R11, R12, R20, R21, R22,
         t0, t1, t2,
         K00, K01, K02, K10, K11, K12, K20, K21, K22) = par

        def proj_body(g, carry):
            sl = pl.ds(g * _L, _L)
            x, y, z = xs_v[sl], ys_v[sl], zs_v[sl]
            px = R00 * x + R01 * y + R02 * z + t0
            py = R10 * x + R11 * y + R12 * z + t1
            pz = R20 * x + R21 * y + R22 * z + t2
            w0 = K00 * px + K01 * py + K02 * pz
            w1 = K10 * px + K11 * py + K12 * pz
            w2 = K20 * px + K21 * py + K22 * pz
            u = jnp.clip(w0 / w2, -65536.0, 65536.0)
            v = jnp.clip(w1 / w2, -65536.0, 65536.0)
            col = jnp.clip(_rhe_int(u) - 1, 0, W - 1)
            row = jnp.clip(_rhe_int(v) - 1, 0, H - 1)
            idx_v[g // 5, pl.ds((g % 5) * _L, _L)] = row * W + col
            px_v[sl], py_v[sl], pz_v[sl] = px, py, pz
            return carry

        lax.fori_loop(0, G, proj_body, 0)

        copies = []
        for tab, dst in ((fmt_hbm, fm_v), (gxt_hbm, gx_v), (gyt_hbm, gy_v)):
            for j in range(2):
                copies.append(pltpu.async_copy(
                    tab.at[idx_v.at[j]], dst.at[pl.ds(j * 80, 80)], sem))
        for cp in copies:
            cp.wait()

        fx, fy = par[12], par[16]
        iot = lax.iota(i32, _L)
        zero = jnp.zeros((_L,), f32)
        gacc = [zero] * 6
        hacc = [zero] * 21
        eacc = zero
        for g in range(G):
            gl = pl.ds(g * _L, _L)
            rows_i = g * _L + iot

            def cbody(c, accs):
                ee, sx, sy, mxx, mxy, myy = accs
                cols_i = jnp.full((_L,), 0, dtype=i32) + c
                fmv = plsc.load_gather(fm_v, [rows_i, cols_i])
                gxv = plsc.load_gather(gx_v, [rows_i, cols_i])
                gyv = plsc.load_gather(gy_v, [rows_i, cols_i])
                frv = plsc.load_gather(fr_v, [rows_i, cols_i])
                e = fmv - frv
                return (ee + e * e, sx + gxv * e, sy + gyv * e,
                        mxx + gxv * gxv, mxy + gxv * gyv, myy + gyv * gyv)

            ee, sx, sy, mxx, mxy, myy = lax.fori_loop(
                0, C, cbody, (zero, zero, zero, zero, zero, zero))
            msk = (base + g * _L + iot) < N
            ee = jnp.where(msk, ee, 0.0)
            sx = jnp.where(msk, sx, 0.0)
            sy = jnp.where(msk, sy, 0.0)
            mxx = jnp.where(msk, mxx, 0.0)
            mxy = jnp.where(msk, mxy, 0.0)
            myy = jnp.where(msk, myy, 0.0)
            eacc = eacc + ee

            px, py, pz = px_v[gl], py_v[gl], pz_v[gl]
            iz = 1.0 / pz
            a = fx * iz
            b = fy * iz
            xz = px * iz
            yz = py * iz
            u_ = [a, zero, -a * xz, -a * px * yz, fx + a * px * xz, -a * py]
            v_ = [zero, b, -b * yz, -fy - b * py * yz, b * px * yz, b * px]
            for j in range(6):
                gacc[j] = gacc[j] + sx * u_[j] + sy * v_[j]
            pj = [mxx * u_[j] + mxy * v_[j] for j in range(6)]
            qj = [mxy * u_[j] + myy * v_[j] for j in range(6)]
            k_ = 0
            for j in range(6):
                for kk in range(j, 6):
                    hacc[k_] = hacc[k_] + u_[kk] * pj[j] + v_[kk] * qj[j]
                    k_ += 1

        for j in range(6):
            st_v[j] = gacc[j]
        for j in range(21):
            st_v[6 + j] = hacc[j]
        st_v[27] = eacc
        pltpu.sync_copy(st_v, out_hbm.at[wid])

    return gn_step


def _skew(v):
    x, y, z = v[..., 0], v[..., 1], v[..., 2]
    o = jnp.zeros_like(x)
    M = jnp.stack([o, -z, y, z, o, -x, -y, x, o], axis=-1)
    return M.reshape(v.shape[:-1] + (3, 3))


def _so3exp(w):
    theta2 = jnp.sum(w * w)
    theta = jnp.sqrt(theta2 + 1e-12)
    W = _skew(w)
    A = jnp.sin(theta) / theta
    B = (1.0 - jnp.cos(theta)) / (theta2 + 1e-12)
    return jnp.eye(3, dtype=w.dtype) + A * W + B * (W @ W)


def _inv6(A):
    B = jnp.concatenate([A, jnp.eye(6, dtype=A.dtype)], axis=1)
    for k in range(6):
        Bk = B[k] / B[k, k]
        B = B - B[:, k:k + 1] * Bk[None, :]
        B = B.at[k].set(Bk)
    return B[:, 6:]


def _opt_step(g, H, lambda_, lr):
    D = jnp.diag(jnp.diag(H) + 1e-9)
    Hd = H + D * lambda_
    P = _inv6(Hd)
    return -lr * (P @ g[..., None])[..., 0]


_ONES16 = jnp.ones((16,), jnp.float32)
_ONES32 = jnp.ones((32,), jnp.float32)
_TRIU = [(j, k) for j in range(6) for k in range(j, 6)]
_TRIU_POS = {jk: i for i, jk in enumerate(_TRIU)}
_HPERM = _np.array([[6 + _TRIU_POS[(min(j, k), max(j, k))] for k in range(6)]
                    for j in range(6)], dtype=_np.int32)


def kernel(pts3D, feature_ref, feature_map_query, feature_grad_x, feature_grad_y, K):
    N, C = feature_ref.shape
    Cm, H, W = feature_map_query.shape
    HW = H * W
    N_pad = ((N + 8 * _NW - 1) // (8 * _NW)) * (8 * _NW)
    gn_step = _build_gn_kernel(N, Cm, H, W, N_pad)

    fm_t = feature_map_query.reshape(Cm, HW).T
    gx_t = feature_grad_x.reshape(Cm, HW).T
    gy_t = feature_grad_y.reshape(Cm, HW).T
    xs = jnp.pad(pts3D[:, 0], (0, N_pad - N))
    ys = jnp.pad(pts3D[:, 1], (0, N_pad - N))
    zs = jnp.pad(pts3D[:, 2], (0, N_pad - N))
    fref_p = jnp.pad(feature_ref, ((0, N_pad - N), (0, 0)))

    def pack(R, t):
        p = jnp.concatenate([R.reshape(9), t, K.reshape(9)])
        return jnp.pad(p, (0, 32 - 21)).astype(jnp.float32)

    dtype = pts3D.dtype
    R = jnp.eye(3, dtype=dtype)
    t = jnp.array([1.0, 1.0, 0.0], dtype=dtype)
    lambda_ = jnp.asarray(LAMBDA_INIT, dtype=dtype)
    lr = jnp.asarray(0.1, dtype=dtype)
    lr_reset = 0.1
    prev_cost = None
    for i in range(N_ITERS):
        part = gn_step(pack(R, t), xs, ys, zs, fref_p, fm_t, gx_t, gy_t)
        sums = _ONES32 @ (part @ _ONES16)
        Grad = sums[0:6]
        Hess = sums[_HPERM]
        if i == 0:
            prev_cost = 0.5 * sums[27] / N
        delta = _opt_step(Grad, Hess, lambda_, lr)
        dt, dw = delta[..., :3], delta[..., 3:6]
        dr = _so3exp(dw)
        R_new = dr @ R
        t_new = dr @ t + dt
        part2 = gn_step(pack(R_new, t_new), xs, ys, zs, fref_p, fm_t, gx_t,
                        gy_t)
        new_cost = 0.5 * (_ONES32 @ (part2[:, 27, :] @ _ONES16)) / N
        increased = new_cost > prev_cost
        lambda_ = jnp.clip(lambda_ * jnp.where(increased, 10.0, 0.1), 1e-6, 1e4)
        lr = jnp.where(increased, jnp.clip(0.1 * lr, 1e-3, 1.0), lr_reset)
        R = jnp.where(increased, R, R_new)
        t = jnp.where(increased, t, t_new)
        prev_cost = jnp.where(increased, prev_cost, new_cost)
    return R, t


# speedup vs baseline: 1.1998x; 1.1998x over previous
import functools

import jax
import jax.numpy as jnp
import numpy as _np
from jax import lax
from jax.experimental import pallas as pl
from jax.experimental.pallas import tpu as pltpu, tpu_sc as plsc

N_ITERS = 3
LAMBDA_INIT = 0.01

_info = plsc.get_sparse_core_info()
_NC, _NS = _info.num_cores, _info.num_subcores
_NW = _NC * _NS
_L = 16


def _rhe_int(u):
    uh = u + 0.5
    i = uh.astype(jnp.int32)
    fi = i.astype(jnp.float32)
    fl = jnp.where(fi > uh, fi - 1.0, fi)
    ifl = fl.astype(jnp.int32)
    half = (fl - u) == 0.5
    odd = (ifl & 1) == 1
    return jnp.where(half & odd, ifl - 1, ifl)


def _build_gn_kernel(N, C, H, W, N_pad):
    HW = H * W
    PPT = N_pad // _NW
    G = PPT // _L
    CK = C // _L
    mesh = plsc.VectorSubcoreMesh(core_axis_name="c", subcore_axis_name="s")
    f32, i32 = jnp.float32, jnp.int32

    @functools.partial(
        pl.kernel, mesh=mesh,
        compiler_params=pltpu.CompilerParams(
            use_tc_tiling_on_sc=False, needs_layout_passes=False),
        out_type=jax.ShapeDtypeStruct((_NW, 28, _L), f32),
        scratch_types=[
            pltpu.VMEM((32,), f32),
            pltpu.VMEM((PPT,), f32),
            pltpu.VMEM((PPT,), f32),
            pltpu.VMEM((PPT,), f32),
            pltpu.VMEM((2, 80), i32),
            pltpu.VMEM((PPT,), f32),
            pltpu.VMEM((PPT,), f32),
            pltpu.VMEM((PPT,), f32),
            pltpu.VMEM((PPT, C), f32),
            pltpu.VMEM((PPT, C), f32),
            pltpu.VMEM((PPT, C), f32),
            pltpu.VMEM((PPT, C), f32),
            pltpu.VMEM((28, _L), f32),
            pltpu.SemaphoreType.DMA,
        ],
    )
    def gn_step(par_hbm, xs_hbm, ys_hbm, zs_hbm, fref_hbm, fmt_hbm, gxt_hbm,
                gyt_hbm, out_hbm, par_v, xs_v, ys_v, zs_v, idx_v, px_v, py_v,
                pz_v, fm_v, gx_v, gy_v, fr_v, st_v, sem):
        wid = lax.axis_index("s") * _NC + lax.axis_index("c")
        base = wid * PPT
        pltpu.sync_copy(par_hbm, par_v)
        pltpu.sync_copy(xs_hbm.at[pl.ds(base, PPT)], xs_v)
        pltpu.sync_copy(ys_hbm.at[pl.ds(base, PPT)], ys_v)
        pltpu.sync_copy(zs_hbm.at[pl.ds(base, PPT)], zs_v)
        pltpu.sync_copy(fref_hbm.at[pl.ds(base, PPT)], fr_v)

        p0 = par_v[pl.ds(0, _L)]
        p1 = par_v[pl.ds(_L, _L)]
        par = [p0[i] for i in range(_L)] + [p1[i] for i in range(5)]
        (R00, R01, R02, R10, R11, R12, R20, R21, R22,
         t0, t1, t2,
         K00, K01, K02, K10, K11, K12, K20, K21, K22) = par

        def proj_body(g, carry):
            sl = pl.ds(g * _L, _L)
            x, y, z = xs_v[sl], ys_v[sl], zs_v[sl]
            px = R00 * x + R01 * y + R02 * z + t0
            py = R10 * x + R11 * y + R12 * z + t1
            pz = R20 * x + R21 * y + R22 * z + t2
            w0 = K00 * px + K01 * py + K02 * pz
            w1 = K10 * px + K11 * py + K12 * pz
            w2 = K20 * px + K21 * py + K22 * pz
            u = jnp.clip(w0 / w2, -65536.0, 65536.0)
            v = jnp.clip(w1 / w2, -65536.0, 65536.0)
            col = jnp.clip(_rhe_int(u) - 1, 0, W - 1)
            row = jnp.clip(_rhe_int(v) - 1, 0, H - 1)
            idx_v[g // 5, pl.ds((g % 5) * _L, _L)] = row * W + col
            px_v[sl], py_v[sl], pz_v[sl] = px, py, pz
            return carry

        lax.fori_loop(0, G, proj_body, 0)

        copies = []
        for tab, dst in ((fmt_hbm, fm_v), (gxt_hbm, gx_v), (gyt_hbm, gy_v)):
            for j in range(2):
                copies.append(pltpu.async_copy(
                    tab.at[idx_v.at[j]], dst.at[pl.ds(j * 80, 80)], sem))
        for cp in copies:
            cp.wait()

        fx, fy = par[12], par[16]
        zero = jnp.zeros((_L,), f32)

        def group_body(g, accs):
            gacc, hacc, eacc = accs
            gl = pl.ds(g * _L, _L)
            pxc, pyc = px_v[gl], py_v[gl]
            izc = 1.0 / pz_v[gl]
            gbase = g * _L
            for i in range(_L):
                p = gbase + i
                ee = zero
                sx = zero
                sy = zero
                mxx = zero
                mxy = zero
                myy = zero
                for k in range(CK):
                    cl = pl.ds(k * _L, _L)
                    e = fm_v[p, cl] - fr_v[p, cl]
                    gxv = gx_v[p, cl]
                    gyv = gy_v[p, cl]
                    ee = ee + e * e
                    sx = sx + gxv * e
                    sy = sy + gyv * e
                    mxx = mxx + gxv * gxv
                    mxy = mxy + gxv * gyv
                    myy = myy + gyv * gyv
                valid = (base + p) < N
                ee = jnp.where(valid, ee, 0.0)
                sx = jnp.where(valid, sx, 0.0)
                sy = jnp.where(valid, sy, 0.0)
                mxx = jnp.where(valid, mxx, 0.0)
                mxy = jnp.where(valid, mxy, 0.0)
                myy = jnp.where(valid, myy, 0.0)
                eacc = eacc + ee
                px, py = pxc[i], pyc[i]
                iz = izc[i]
                a = fx * iz
                b = fy * iz
                xz = px * iz
                yz = py * iz
                u_ = [a, None, -a * xz, -a * px * yz, fx + a * px * xz,
                      -a * py]
                v_ = [None, b, -b * yz, -fy - b * py * yz, b * px * yz,
                      b * px]
                gacc = list(gacc)
                hacc = list(hacc)
                for j in range(6):
                    if u_[j] is None:
                        gacc[j] = gacc[j] + sy * v_[j]
                    elif v_[j] is None:
                        gacc[j] = gacc[j] + sx * u_[j]
                    else:
                        gacc[j] = gacc[j] + sx * u_[j] + sy * v_[j]
                pj = []
                qj = []
                for j in range(6):
                    uj = 0.0 if u_[j] is None else u_[j]
                    vj = 0.0 if v_[j] is None else v_[j]
                    pj.append(mxx * uj + mxy * vj)
                    qj.append(mxy * uj + myy * vj)
                k_ = 0
                for j in range(6):
                    for kk in range(j, 6):
                        uk = 0.0 if u_[kk] is None else u_[kk]
                        vk = 0.0 if v_[kk] is None else v_[kk]
                        hacc[k_] = hacc[k_] + uk * pj[j] + vk * qj[j]
                        k_ += 1
            return (tuple(gacc), tuple(hacc), eacc)

        gacc, hacc, eacc = lax.fori_loop(
            0, G, group_body, ((zero,) * 6, (zero,) * 21, zero))
        for j in range(6):
            st_v[j] = gacc[j]
        for j in range(21):
            st_v[6 + j] = hacc[j]
        st_v[27] = eacc
        pltpu.sync_copy(st_v, out_hbm.at[wid])

    return gn_step


def _skew(v):
    x, y, z = v[..., 0], v[..., 1], v[..., 2]
    o = jnp.zeros_like(x)
    M = jnp.stack([o, -z, y, z, o, -x, -y, x, o], axis=-1)
    return M.reshape(v.shape[:-1] + (3, 3))


def _so3exp(w):
    theta2 = jnp.sum(w * w)
    theta = jnp.sqrt(theta2 + 1e-12)
    W = _skew(w)
    A = jnp.sin(theta) / theta
    B = (1.0 - jnp.cos(theta)) / (theta2 + 1e-12)
    return jnp.eye(3, dtype=w.dtype) + A * W + B * (W @ W)


def _inv6(A):
    B = jnp.concatenate([A, jnp.eye(6, dtype=A.dtype)], axis=1)
    for k in range(6):
        Bk = B[k] / B[k, k]
        B = B - B[:, k:k + 1] * Bk[None, :]
        B = B.at[k].set(Bk)
    return B[:, 6:]


def _opt_step(g, H, lambda_, lr):
    D = jnp.diag(jnp.diag(H) + 1e-9)
    Hd = H + D * lambda_
    P = _inv6(Hd)
    return -lr * (P @ g[..., None])[..., 0]


_ONES16 = jnp.ones((16,), jnp.float32)
_ONES32 = jnp.ones((32,), jnp.float32)
_TRIU = [(j, k) for j in range(6) for k in range(j, 6)]
_TRIU_POS = {jk: i for i, jk in enumerate(_TRIU)}
_HPERM = _np.array([[6 + _TRIU_POS[(min(j, k), max(j, k))] for k in range(6)]
                    for j in range(6)], dtype=_np.int32)


def kernel(pts3D, feature_ref, feature_map_query, feature_grad_x, feature_grad_y, K):
    N, C = feature_ref.shape
    Cm, H, W = feature_map_query.shape
    HW = H * W
    N_pad = ((N + 8 * _NW - 1) // (8 * _NW)) * (8 * _NW)
    gn_step = _build_gn_kernel(N, Cm, H, W, N_pad)

    fm_t = feature_map_query.reshape(Cm, HW).T
    gx_t = feature_grad_x.reshape(Cm, HW).T
    gy_t = feature_grad_y.reshape(Cm, HW).T
    xs = jnp.pad(pts3D[:, 0], (0, N_pad - N))
    ys = jnp.pad(pts3D[:, 1], (0, N_pad - N))
    zs = jnp.pad(pts3D[:, 2], (0, N_pad - N))
    fref_p = jnp.pad(feature_ref, ((0, N_pad - N), (0, 0)))

    def pack(R, t):
        p = jnp.concatenate([R.reshape(9), t, K.reshape(9)])
        return jnp.pad(p, (0, 32 - 21)).astype(jnp.float32)

    dtype = pts3D.dtype
    f32 = jnp.float32
    R0 = jnp.eye(3, dtype=dtype)
    t0 = jnp.array([1.0, 1.0, 0.0], dtype=dtype)

    def step(carry, sidx):
        R, t, R_new, t_new, lambda_, lr, prev_cost = carry
        phase_a = (sidx % 2) == 0
        Ru = jnp.where(phase_a, R, R_new)
        tu = jnp.where(phase_a, t, t_new)
        part = gn_step(pack(Ru, tu), xs, ys, zs, fref_p, fm_t, gx_t, gy_t)
        sums = _ONES32 @ (part @ _ONES16)
        cost = 0.5 * sums[27] / N
        prev_cost_a = jnp.where(sidx == 0, cost, prev_cost)
        Grad = sums[0:6]
        Hess = sums[_HPERM]
        delta = _opt_step(Grad, Hess, lambda_, lr)
        dr = _so3exp(delta[3:6])
        R_new_a = dr @ R
        t_new_a = dr @ t + delta[:3]
        increased = cost > prev_cost
        lambda_b = jnp.clip(lambda_ * jnp.where(increased, 10.0, 0.1),
                            1e-6, 1e4)
        lr_b = jnp.where(increased, jnp.clip(0.1 * lr, 1e-3, 1.0), 0.1)
        R_b = jnp.where(increased, R, R_new)
        t_b = jnp.where(increased, t, t_new)
        prev_cost_b = jnp.where(increased, prev_cost, cost)
        carry = (
            jnp.where(phase_a, R, R_b),
            jnp.where(phase_a, t, t_b),
            jnp.where(phase_a, R_new_a, R_new),
            jnp.where(phase_a, t_new_a, t_new),
            jnp.where(phase_a, lambda_, lambda_b),
            jnp.where(phase_a, lr, lr_b),
            jnp.where(phase_a, prev_cost_a, prev_cost_b),
        )
        return carry, jnp.float32(0.0)

    init = (R0, t0, R0, t0,
            jnp.asarray(LAMBDA_INIT, dtype=dtype),
            jnp.asarray(0.1, dtype=dtype),
            jnp.asarray(0.0, dtype=f32))
    carry, _ = lax.scan(step, init, jnp.arange(2 * N_ITERS))
    R, t = carry[0], carry[1]
    return R, t
